# baseline (speedup 1.0000x reference)
import functools

import jax
import jax.numpy as jnp
from jax import lax
from jax.experimental import pallas as pl
from jax.experimental.pallas import tpu as pltpu
from jax.experimental.pallas import tpu_sc as plsc

N, D = 32768, 512
B, L = 16, 4096

NC, NS, LANES = 2, 16, 16
NW = NC * NS
RPW = N // NW
CH = 32
NCH = RPW // CH
HALF = N // NC
IDX_PER = (B * L) // NS
GROUPS = CH // LANES
DJ = D // LANES
NBUF = 4


def _fused_body(x_hbm, w_hbm, bagsT_hbm, out_hbm,
                xbuf0, xbuf1, xbuf2, xbuf3, w_v, idx_v, accmat_v, mylog_v,
                halflog_v, part_v, half_sh, sem0, sem1, sem2, sem3):
    c = lax.axis_index("c")
    s = lax.axis_index("s")
    rowbase = (c * NS + s) * RPW

    pltpu.sync_copy(w_hbm, w_v)
    pltpu.sync_copy(bagsT_hbm.at[pl.ds(s * IDX_PER, IDX_PER)], idx_v)

    xb = (xbuf0, xbuf1, xbuf2, xbuf3)
    sems = (sem0, sem1, sem2, sem3)
    for p in range(NBUF):
        pltpu.async_copy(x_hbm.at[pl.ds(rowbase + p * CH, CH), :], xb[p], sems[p])

    wregs = [w_v[pl.ds(j * LANES, LANES)] for j in range(DJ)]

    def make_group(buf, k):
        def group(g, carry):
            for i in range(LANES):
                row = g * LANES + i
                prods = [xb[buf][row, pl.ds(j * LANES, LANES)] * wregs[j]
                         for j in range(DJ)]
                while len(prods) > 1:
                    prods = [prods[m] + prods[m + 1]
                             for m in range(0, len(prods), 2)]
                accmat_v[pl.ds(i * LANES, LANES)] = prods[0]
            lanes = lax.iota(jnp.int32, LANES) * LANES
            gath = [plsc.load_gather(accmat_v, [lanes + t]) for t in range(LANES)]
            while len(gath) > 1:
                gath = [gath[m] + gath[m + 1] for m in range(0, len(gath), 2)]
            mylog_v[pl.ds(k * CH + g * LANES, LANES)] = gath[0]
            return carry
        return group

    def chunk_iter(it, carry):
        for bpar in range(NBUF):
            k = it * NBUF + bpar
            pltpu.make_async_copy(
                x_hbm.at[pl.ds(rowbase, CH), :], xb[bpar], sems[bpar]).wait()
            lax.fori_loop(0, GROUPS, make_group(bpar, k), 0)

            @pl.when(k + NBUF < NCH)
            def _():
                pltpu.async_copy(
                    x_hbm.at[pl.ds(rowbase + (k + NBUF) * CH, CH), :],
                    xb[bpar], sems[bpar])
        return carry

    lax.fori_loop(0, NCH // NBUF, chunk_iter, 0)

    pltpu.sync_copy(mylog_v, half_sh.at[pl.ds(s * RPW, RPW)])
    plsc.subcore_barrier()
    pltpu.sync_copy(half_sh, halflog_v)

    base = c * HALF

    def gbody(j, acc):
        idx = idx_v[pl.ds(j * LANES, LANES)]
        local = idx - base
        valid = (local >= 0) & (local < HALF)
        clamped = jnp.minimum(jnp.maximum(local, 0), HALF - 1)
        vals = plsc.load_gather(halflog_v, [clamped])
        vals = jnp.where(valid, vals, -jnp.inf)
        return jnp.maximum(acc, vals)

    acc = lax.fori_loop(0, IDX_PER // LANES, gbody,
                        jnp.full((LANES,), -jnp.inf, jnp.float32))

    part_v[...] = acc
    pltpu.sync_copy(part_v, out_hbm.at[c * NS + s])


_fused = functools.partial(
    pl.kernel,
    out_type=jax.ShapeDtypeStruct((NW, LANES), jnp.float32),
    mesh=plsc.VectorSubcoreMesh(
        core_axis_name="c", subcore_axis_name="s",
        num_cores=NC, num_subcores=NS),
    compiler_params=pltpu.CompilerParams(needs_layout_passes=False),
    scratch_types=[
        pltpu.VMEM((CH, D), jnp.float32),
        pltpu.VMEM((CH, D), jnp.float32),
        pltpu.VMEM((CH, D), jnp.float32),
        pltpu.VMEM((CH, D), jnp.float32),
        pltpu.VMEM((D,), jnp.float32),
        pltpu.VMEM((IDX_PER,), jnp.int32),
        pltpu.VMEM((LANES * LANES,), jnp.float32),
        pltpu.VMEM((RPW,), jnp.float32),
        pltpu.VMEM((HALF,), jnp.float32),
        pltpu.VMEM((LANES,), jnp.float32),
        pltpu.VMEM_SHARED((HALF,), jnp.float32),
        pltpu.SemaphoreType.DMA,
        pltpu.SemaphoreType.DMA,
        pltpu.SemaphoreType.DMA,
        pltpu.SemaphoreType.DMA,
    ],
)(_fused_body)


def kernel(X, bags, bags_mask, W, b):
    bagsT = bags.T.reshape(L * B)
    part = _fused(X, W.reshape(D), bagsT)
    m = (jnp.max(part, axis=0) + b[0]).reshape(B, 1)
    p = jax.nn.sigmoid(m)
    return jnp.log(jnp.concatenate([1.0 - p, p], axis=1))


# speedup vs baseline: 1.3293x; 1.3293x over previous
import functools

import jax
import jax.numpy as jnp
from jax import lax
from jax.experimental import pallas as pl
from jax.experimental.pallas import tpu as pltpu
from jax.experimental.pallas import tpu_sc as plsc

N, D = 32768, 512
B, L = 16, 4096

NC, NS, LANES = 2, 16, 16
NW = NC * NS
ROWS_PER_W = L // NW

BN = 4096


def _matvec_body(x_ref, wt_ref, b_ref, o_ref):
    o_ref[...] = jnp.sum(x_ref[...] * wt_ref[...], axis=1, keepdims=True) + b_ref[0]


def _matvec(X, W, b):
    return pl.pallas_call(
        _matvec_body,
        grid=(N // BN,),
        in_specs=[
            pl.BlockSpec((BN, D), lambda i: (i, 0)),
            pl.BlockSpec((1, D), lambda i: (0, 0)),
            pl.BlockSpec(memory_space=pltpu.SMEM),
        ],
        out_specs=pl.BlockSpec((BN, 1), lambda i: (i, 0)),
        out_shape=jax.ShapeDtypeStruct((N, 1), jnp.float32),
    )(X, W.reshape(1, D), b)


HALF = N // NC
COLS = L // NS


def _bag_max_body(logits_hbm, bags_hbm, out_hbm, halflog_v, bchunk_v, part_v):
    c = lax.axis_index("c")
    s = lax.axis_index("s")

    pltpu.sync_copy(logits_hbm.at[pl.ds(c * HALF, HALF)], halflog_v)
    for bb in range(B):
        pltpu.sync_copy(bags_hbm.at[bb, pl.ds(s * COLS, COLS)],
                        bchunk_v.at[pl.ds(bb * COLS, COLS)])

    base = c * HALF
    lanes = lax.iota(jnp.int32, LANES) * COLS

    def body(j, acc):
        idx = plsc.load_gather(bchunk_v, [lanes + j])
        local = idx - base
        valid = (local >= 0) & (local < HALF)
        clamped = jnp.minimum(jnp.maximum(local, 0), HALF - 1)
        vals = plsc.load_gather(halflog_v, [clamped])
        vals = jnp.where(valid, vals, -jnp.inf)
        return jnp.maximum(acc, vals)

    acc = lax.fori_loop(0, COLS, body,
                        jnp.full((LANES,), -jnp.inf, jnp.float32))

    part_v[...] = acc
    pltpu.sync_copy(part_v, out_hbm.at[c * NS + s])


_bag_max = functools.partial(
    pl.kernel,
    out_type=jax.ShapeDtypeStruct((NW, LANES), jnp.float32),
    mesh=plsc.VectorSubcoreMesh(
        core_axis_name="c", subcore_axis_name="s",
        num_cores=NC, num_subcores=NS),
    compiler_params=pltpu.CompilerParams(needs_layout_passes=False),
    scratch_types=[
        pltpu.VMEM((HALF,), jnp.float32),
        pltpu.VMEM((B * COLS,), jnp.int32),
        pltpu.VMEM((LANES,), jnp.float32),
    ],
)(_bag_max_body)


def kernel(X, bags, bags_mask, W, b):
    logits = _matvec(X, W, b).reshape(N)
    per_core = _bag_max(logits, bags)
    m = jnp.max(per_core, axis=0).reshape(B, 1)
    p = jax.nn.sigmoid(m)
    return jnp.log(jnp.concatenate([1.0 - p, p], axis=1))


# speedup vs baseline: 1.6465x; 1.2386x over previous
import functools

import jax
import jax.numpy as jnp
from jax import lax
from jax.experimental import pallas as pl
from jax.experimental.pallas import tpu as pltpu
from jax.experimental.pallas import tpu_sc as plsc

N, D = 32768, 512
B, L = 16, 4096

NC, NS, LANES = 2, 16, 16
NW = NC * NS
ROWS_PER_W = L // NW

BN = 4096


def _matvec_body(x_ref, wt_ref, b_ref, o_ref):
    o_ref[...] = jnp.sum(x_ref[...] * wt_ref[...], axis=1) + b_ref[0]


def _matvec(X, W, b):
    return pl.pallas_call(
        _matvec_body,
        grid=(N // BN,),
        in_specs=[
            pl.BlockSpec((BN, D), lambda i: (i, 0)),
            pl.BlockSpec((1, D), lambda i: (0, 0)),
            pl.BlockSpec(memory_space=pltpu.SMEM),
        ],
        out_specs=pl.BlockSpec((BN,), lambda i: (i,)),
        out_shape=jax.ShapeDtypeStruct((N,), jnp.float32),
    )(X, W.reshape(1, D), b)


def _bag_max_body(logits_hbm, bagsT_hbm, out_hbm, logits_v, idx_v, part_v):
    c = lax.axis_index("c")
    s = lax.axis_index("s")
    wid = s * NC + c

    pltpu.sync_copy(logits_hbm, logits_v)
    chunk = ROWS_PER_W * LANES
    pltpu.sync_copy(bagsT_hbm.at[pl.ds(wid * chunk, chunk)], idx_v)

    def body(j, acc):
        idx = idx_v[pl.ds(j * LANES, LANES)]
        vals = plsc.load_gather(logits_v, [idx])
        return jnp.maximum(acc, vals)

    acc = lax.fori_loop(0, ROWS_PER_W, body,
                        jnp.full((LANES,), -jnp.inf, jnp.float32))

    part_v[...] = acc
    pltpu.sync_copy(part_v, out_hbm.at[wid])


_bag_max = functools.partial(
    pl.kernel,
    out_type=jax.ShapeDtypeStruct((NW, LANES), jnp.float32),
    mesh=plsc.VectorSubcoreMesh(
        core_axis_name="c", subcore_axis_name="s",
        num_cores=NC, num_subcores=NS),
    compiler_params=pltpu.CompilerParams(needs_layout_passes=False),
    scratch_types=[
        pltpu.VMEM((N,), jnp.float32),
        pltpu.VMEM((ROWS_PER_W * LANES,), jnp.int32),
        pltpu.VMEM((LANES,), jnp.float32),
    ],
)(_bag_max_body)


def kernel(X, bags, bags_mask, W, b):
    logits = _matvec(X, W, b)
    bagsT = bags.T.reshape(L * B)
    per_core = _bag_max(logits, bagsT)
    m = jnp.max(per_core, axis=0).reshape(B, 1)
    p = jax.nn.sigmoid(m)
    return jnp.log(jnp.concatenate([1.0 - p, p], axis=1))
